# baseline (speedup 1.0000x reference)
import jax
import jax.numpy as jnp
from jax import lax
from jax.experimental import pallas as pl
from jax.experimental.pallas import tpu as pltpu
from jax.experimental.pallas import tpu_sc as plsc

D = 64
LANE = 128
NC, NS = 2, 16
NW = NC * NS
NB = 5


def _emb_body(idx_hbm, w_hbm, out_hbm, idx_v, rows_v, gsem, osem):
    tok_rows = idx_hbm.shape[0]
    rpw = tok_rows // NW

    c = lax.axis_index("c")
    s = lax.axis_index("s")
    wid = s * NC + c
    base = wid * rpw

    def start(g, b, first=False):
        pltpu.sync_copy(idx_hbm.at[pl.ds(base + g, 1)],
                        idx_v.at[pl.ds(b, 1)])
        if not first:
            pltpu.make_async_copy(rows_v.at[pl.ds(b, 1)],
                                  out_hbm.at[pl.ds(0, 1)], osem.at[b]).wait()
        pltpu.async_copy(w_hbm.at[idx_v.at[b]], rows_v.at[b], gsem.at[b])

    def finish(g, b):
        pltpu.make_async_copy(w_hbm.at[pl.ds(0, LANE)], rows_v.at[b],
                              gsem.at[b]).wait()
        pltpu.async_copy(rows_v.at[pl.ds(b, 1)],
                         out_hbm.at[pl.ds(base + g, 1)], osem.at[b])

    for b in range(NB):
        start(b, b, first=True)

    def loop_body(i, carry):
        g = NB * i
        for b in range(NB):
            finish(g + b, b)
            start(g + NB + b, b)
        return carry

    lax.fori_loop(0, rpw // NB - 1, loop_body, 0)
    for b in range(NB):
        finish(rpw - NB + b, b)
    for b in range(NB):
        pltpu.make_async_copy(rows_v.at[pl.ds(b, 1)], out_hbm.at[pl.ds(0, 1)],
                              osem.at[b]).wait()


def kernel(tokens, weight):
    b0, b1 = tokens.shape
    vocab = weight.shape[0]
    tok_rows = (b0 * b1) // LANE
    idx = tokens.reshape(tok_rows, LANE)
    w128 = jnp.broadcast_to(weight.reshape(vocab, 1, D),
                            (vocab, LANE // D, D)).reshape(vocab, LANE)
    mesh = plsc.VectorSubcoreMesh(core_axis_name="c", subcore_axis_name="s",
                                  num_cores=NC, num_subcores=NS)
    out = pl.kernel(
        _emb_body,
        out_type=jax.ShapeDtypeStruct((tok_rows, LANE, LANE), jnp.float32),
        mesh=mesh,
        scratch_types=[
            pltpu.VMEM((NB, LANE), jnp.int32),
            pltpu.VMEM((NB, LANE, LANE), jnp.float32),
            pltpu.SemaphoreType.DMA((NB,)),
            pltpu.SemaphoreType.DMA((NB,)),
        ],
    )(idx, w128)
    return out[:, :, :D].reshape(b0, b1, D)


# speedup vs baseline: 1.1557x; 1.1557x over previous
import jax
import jax.numpy as jnp
from jax import lax
from jax.experimental import pallas as pl
from jax.experimental.pallas import tpu as pltpu
from jax.experimental.pallas import tpu_sc as plsc

D = 64
LANE = 128
NC, NS = 2, 16
NW = NC * NS
NB = 5


def _emb_body(idx_hbm, w_hbm, out_hbm, idx_v, rows_v, gsem, osem):
    tok_rows = idx_hbm.shape[0]
    rpw = tok_rows // NW

    c = lax.axis_index("c")
    s = lax.axis_index("s")
    wid = s * NC + c
    base = wid * rpw

    def start(g, b, first=False):
        pltpu.sync_copy(idx_hbm.at[pl.ds(base + g, 1)],
                        idx_v.at[pl.ds(b, 1)])
        if not first:
            pltpu.make_async_copy(rows_v.at[pl.ds(b, 1)],
                                  out_hbm.at[pl.ds(0, 1)], osem.at[b]).wait()
        pltpu.async_copy(w_hbm.at[idx_v.at[b]], rows_v.at[b], gsem.at[b])

    def finish(g, b):
        pltpu.make_async_copy(w_hbm.at[pl.ds(0, LANE)], rows_v.at[b],
                              gsem.at[b]).wait()
        pltpu.async_copy(rows_v.at[pl.ds(b, 1)],
                         out_hbm.at[pl.ds(base + g, 1)], osem.at[b])

    for b in range(NB):
        start(b, b, first=True)

    def loop_body(i, carry):
        g = NB * i
        for b in range(NB):
            finish(g + b, b)
            start(g + NB + b, b)
        return carry

    lax.fori_loop(0, rpw // NB - 1, loop_body, 0)
    for b in range(NB):
        finish(rpw - NB + b, b)
    for b in range(NB):
        pltpu.make_async_copy(rows_v.at[pl.ds(b, 1)], out_hbm.at[pl.ds(0, 1)],
                              osem.at[b]).wait()


def kernel(tokens, weight):
    b0, b1 = tokens.shape
    vocab = weight.shape[0]
    tok_rows = (b0 * b1) // LANE
    idx = tokens.reshape(tok_rows, LANE)
    w128 = jnp.pad(weight, ((0, 0), (0, LANE - D)))
    mesh = plsc.VectorSubcoreMesh(core_axis_name="c", subcore_axis_name="s",
                                  num_cores=NC, num_subcores=NS)
    out = pl.kernel(
        _emb_body,
        out_type=jax.ShapeDtypeStruct((tok_rows, LANE, LANE), jnp.float32),
        mesh=mesh,
        scratch_types=[
            pltpu.VMEM((NB, LANE), jnp.int32),
            pltpu.VMEM((NB, LANE, LANE), jnp.float32),
            pltpu.SemaphoreType.DMA((NB,)),
            pltpu.SemaphoreType.DMA((NB,)),
        ],
    )(idx, w128)
    return out[:, :, :D].reshape(b0, b1, D)
